# baseline (speedup 1.0000x reference)
import functools

import jax
import jax.numpy as jnp
from jax import lax
from jax.experimental import pallas as pl
from jax.experimental.pallas import tpu as pltpu
from jax.experimental.pallas import tpu_sc as plsc

_G = 256
_EPS = 1e-9
_L = 16
_NC, _NS = 2, 16
_NW = _NC * _NS
_C = 80


def _worker_id():
  return lax.axis_index("s") * _NC + lax.axis_index("c")


def _sc_mesh():
  return plsc.VectorSubcoreMesh(
      core_axis_name="c", subcore_axis_name="s",
      num_cores=_NC, num_subcores=_NS)


def _stats(x, batch, interpret=False):
  n, d = x.shape
  nf = d // _L
  n_chunks = n // _C
  assert n_chunks * _C == n

  @functools.partial(
      pl.kernel,
      out_type=[
          jax.ShapeDtypeStruct((_NW, _G, d), jnp.float32),
          jax.ShapeDtypeStruct((_NW, _G, d), jnp.float32),
          jax.ShapeDtypeStruct((_NW, _G, _L), jnp.float32),
      ],
      mesh=_sc_mesh(),
      scratch_types=[
          pltpu.VMEM((2, _C, d), jnp.float32),
          pltpu.VMEM((_C,), jnp.int32),
          pltpu.VMEM((_C,), jnp.int32),
          pltpu.VMEM((_G, d), jnp.float32),
          pltpu.VMEM((_G, d), jnp.float32),
          pltpu.VMEM((_G, _L), jnp.float32),
          pltpu.SemaphoreType.DMA,
          pltpu.SemaphoreType.DMA,
      ],
      interpret=interpret,
  )
  def k(x_hbm, b_hbm, sum_hbm, sq_hbm, cnt_hbm, xv, iv0, iv1, sumv, sqv,
        cntv, sem0, sem1):
    wid = _worker_id()
    zeros = jnp.zeros((_L,), jnp.float32)
    ones = jnp.ones((_L,), jnp.float32)

    lo = (n_chunks * wid) // _NW
    hi = (n_chunks * (wid + 1)) // _NW

    def in_start(c, b):
      @pl.when(b == 0)
      def _():
        pltpu.async_copy(x_hbm.at[pl.ds(c * _C, _C)], xv.at[0], sem0)
        pltpu.async_copy(b_hbm.at[pl.ds(c * _C, _C)], iv0, sem0)

      @pl.when(b == 1)
      def _():
        pltpu.async_copy(x_hbm.at[pl.ds(c * _C, _C)], xv.at[1], sem1)
        pltpu.async_copy(b_hbm.at[pl.ds(c * _C, _C)], iv1, sem1)

    def in_wait(b):
      @pl.when(b == 0)
      def _():
        pltpu.make_async_copy(x_hbm.at[pl.ds(0, _C)], xv.at[0], sem0).wait()
        pltpu.make_async_copy(b_hbm.at[pl.ds(0, _C)], iv0, sem0).wait()

      @pl.when(b == 1)
      def _():
        pltpu.make_async_copy(x_hbm.at[pl.ds(0, _C)], xv.at[1], sem1).wait()
        pltpu.make_async_copy(b_hbm.at[pl.ds(0, _C)], iv1, sem1).wait()

    def zero_body(g, carry):
      for f in range(nf):
        s = pl.ds(f * _L, _L)
        sumv[g, s] = zeros
        sqv[g, s] = zeros
      cntv[g, :] = zeros
      return carry

    in_start(lo, 0)
    lax.fori_loop(0, _G, zero_body, 0)

    def chunk_body(c, carry):
      b = lax.rem(c - lo, 2)

      @pl.when(c + 1 < hi)
      def _():
        in_start(c + 1, 1 - b)

      in_wait(b)

      def grp_body(q, rc):
        gvec = jnp.where(b == 0, iv0[pl.ds(q * _L, _L)],
                         iv1[pl.ds(q * _L, _L)])
        g0 = gvec[0]
        g15 = gvec[_L - 1]

        @pl.when(g0 == g15)
        def _fast():
          accs = []
          accq = []
          for f in range(nf):
            s = pl.ds(f * _L, _L)
            v = xv[b, q * _L, s]
            accs.append(v)
            accq.append(v * v)
          for j in range(1, _L):
            r = q * _L + j
            for f in range(nf):
              s = pl.ds(f * _L, _L)
              v = xv[b, r, s]
              accs[f] = accs[f] + v
              accq[f] = accq[f] + v * v
          for f in range(nf):
            s = pl.ds(f * _L, _L)
            plsc.addupdate(sumv.at[g0, s], accs[f])
            plsc.addupdate(sqv.at[g0, s], accq[f])
          plsc.addupdate(cntv.at[g0, :], ones * float(_L))

        @pl.when(g0 != g15)
        def _slow():
          for j in range(_L):
            g = gvec[j]
            r = q * _L + j
            for f in range(nf):
              s = pl.ds(f * _L, _L)
              v = xv[b, r, s]
              plsc.addupdate(sumv.at[g, s], v)
              plsc.addupdate(sqv.at[g, s], v * v)
            plsc.addupdate(cntv.at[g, :], ones)

        return rc

      lax.fori_loop(0, _C // _L, grp_body, 0)
      return carry

    lax.fori_loop(lo, hi, chunk_body, 0)
    pltpu.sync_copy(sumv, sum_hbm.at[wid])
    pltpu.sync_copy(sqv, sq_hbm.at[wid])
    pltpu.sync_copy(cntv, cnt_hbm.at[wid])

  return k(x, batch)


def _combine(sum_p, sq_p, cnt_p, alpha, weight, bias, interpret=False):
  d = sum_p.shape[-1]

  def k(sum_ref, sq_ref, cnt_ref, a_ref, w_ref, b_ref, scale_ref, shift_ref):
    sums = jnp.sum(sum_ref[...], axis=0)
    sqs = jnp.sum(sq_ref[...], axis=0)
    cnt = jnp.sum(cnt_ref[...], axis=0)[:, 0:1]
    cnt = jnp.maximum(cnt, 1.0)
    mu = sums / cnt
    m2 = sqs / cnt
    a = a_ref[...]
    w = w_ref[...]
    b = b_ref[...]
    sig2 = m2 - (2.0 * a - a * a) * mu * mu
    sig2 = jnp.maximum(sig2, 0.0) + _EPS
    rstd = lax.rsqrt(sig2)
    scale_ref[...] = w * rstd
    shift_ref[...] = b - w * a * mu * rstd

  return pl.pallas_call(
      k,
      out_shape=[
          jax.ShapeDtypeStruct((_G, d), jnp.float32),
          jax.ShapeDtypeStruct((_G, d), jnp.float32),
      ],
      interpret=interpret,
  )(sum_p, sq_p, cnt_p, alpha, weight, bias)


_CA = 32


def _apply(x, batch, scale, shift, interpret=False):
  n, d = x.shape
  nf = d // _L
  n_chunks = n // _CA
  assert n_chunks * _CA == n

  @functools.partial(
      pl.kernel,
      out_type=jax.ShapeDtypeStruct((n, d), jnp.float32),
      mesh=_sc_mesh(),
      scratch_types=[
          pltpu.VMEM((2, _CA, d), jnp.float32),
          pltpu.VMEM((2, _CA, d), jnp.float32),
          pltpu.VMEM((_CA,), jnp.int32),
          pltpu.VMEM((_CA,), jnp.int32),
          pltpu.VMEM((_G, d), jnp.float32),
          pltpu.VMEM((_G, d), jnp.float32),
          pltpu.SemaphoreType.DMA,
          pltpu.SemaphoreType.DMA,
          pltpu.SemaphoreType.DMA,
          pltpu.SemaphoreType.DMA,
      ],
      interpret=interpret,
  )
  def k(x_hbm, b_hbm, sc_hbm, sh_hbm, out_hbm, xv, ov, iv0, iv1, scv, shv,
        semi0, semi1, semo0, semo1):
    wid = _worker_id()

    lo = (n_chunks * wid) // _NW
    hi = (n_chunks * (wid + 1)) // _NW

    def in_start(c, b):
      @pl.when(b == 0)
      def _():
        pltpu.async_copy(x_hbm.at[pl.ds(c * _CA, _CA)], xv.at[0], semi0)
        pltpu.async_copy(b_hbm.at[pl.ds(c * _CA, _CA)], iv0, semi0)

      @pl.when(b == 1)
      def _():
        pltpu.async_copy(x_hbm.at[pl.ds(c * _CA, _CA)], xv.at[1], semi1)
        pltpu.async_copy(b_hbm.at[pl.ds(c * _CA, _CA)], iv1, semi1)

    def in_wait(b):
      @pl.when(b == 0)
      def _():
        pltpu.make_async_copy(x_hbm.at[pl.ds(0, _CA)], xv.at[0], semi0).wait()
        pltpu.make_async_copy(b_hbm.at[pl.ds(0, _CA)], iv0, semi0).wait()

      @pl.when(b == 1)
      def _():
        pltpu.make_async_copy(x_hbm.at[pl.ds(0, _CA)], xv.at[1], semi1).wait()
        pltpu.make_async_copy(b_hbm.at[pl.ds(0, _CA)], iv1, semi1).wait()

    def out_start(c, b):
      @pl.when(b == 0)
      def _():
        pltpu.async_copy(ov.at[0], out_hbm.at[pl.ds(c * _CA, _CA)], semo0)

      @pl.when(b == 1)
      def _():
        pltpu.async_copy(ov.at[1], out_hbm.at[pl.ds(c * _CA, _CA)], semo1)

    def out_wait(b):
      @pl.when(b == 0)
      def _():
        pltpu.make_async_copy(ov.at[0], out_hbm.at[pl.ds(0, _CA)],
                              semo0).wait()

      @pl.when(b == 1)
      def _():
        pltpu.make_async_copy(ov.at[1], out_hbm.at[pl.ds(0, _CA)],
                              semo1).wait()

    in_start(lo, 0)
    pltpu.sync_copy(sc_hbm, scv)
    pltpu.sync_copy(sh_hbm, shv)

    def chunk_body(c, carry):
      b = lax.rem(c - lo, 2)

      @pl.when(c + 1 < hi)
      def _():
        in_start(c + 1, 1 - b)

      in_wait(b)

      def grp_body(q, rc):
        gvec = jnp.where(b == 0, iv0[pl.ds(q * _L, _L)],
                         iv1[pl.ds(q * _L, _L)])
        g0 = gvec[0]
        g15 = gvec[_L - 1]

        @pl.when(g0 == g15)
        def _fast():
          scr = []
          shr = []
          for f in range(nf):
            s = pl.ds(f * _L, _L)
            scr.append(scv[g0, s])
            shr.append(shv[g0, s])
          for j in range(_L):
            r = q * _L + j
            for f in range(nf):
              s = pl.ds(f * _L, _L)
              ov[b, r, s] = xv[b, r, s] * scr[f] + shr[f]

        @pl.when(g0 != g15)
        def _slow():
          for j in range(_L):
            g = gvec[j]
            r = q * _L + j
            for f in range(nf):
              s = pl.ds(f * _L, _L)
              ov[b, r, s] = xv[b, r, s] * scv[g, s] + shv[g, s]

        return rc

      @pl.when(c - 2 >= lo)
      def _():
        out_wait(b)

      lax.fori_loop(0, _CA // _L, grp_body, 0)
      out_start(c, b)
      return carry

    lax.fori_loop(lo, hi, chunk_body, 0)

    @pl.when(hi - lo >= 2)
    def _():
      out_wait(lax.rem(hi - 2 - lo, 2))

    @pl.when(hi - lo >= 1)
    def _():
      out_wait(lax.rem(hi - 1 - lo, 2))

  return k(x, batch, scale, shift)


def kernel(x, batch, alpha, weight, bias):
  batch = batch.astype(jnp.int32)
  sum_p, sq_p, cnt_p = _stats(x, batch)
  scale, shift = _combine(
      sum_p, sq_p, cnt_p,
      alpha.reshape(1, -1), weight.reshape(1, -1), bias.reshape(1, -1))
  return _apply(x, batch, scale, shift)
